# Initial kernel scaffold; baseline (speedup 1.0000x reference)
import jax
import jax.numpy as jnp
from jax.experimental import pallas as pl


def kernel(static, known_real, known_categorical, observed, static_tables, known_cat_tables, real_W, real_b, obs_W, obs_b):
    raise NotImplementedError("write your pallas kernel here")


# speedup vs baseline: 1.0982x; 1.0982x over previous
import functools

import jax
import jax.numpy as jnp
from jax import lax
from jax.experimental import pallas as pl
from jax.experimental.pallas import tpu as pltpu
from jax.experimental.pallas import tpu_sc as plsc

_NC = 2
_NS = 16
_NW = _NC * _NS
_CHUNK = 128
_G = 10


def _sc_gather(tab_s, tab_c, idx_s, idx_c):
    H = tab_s.shape[-1]
    sr_pw = idx_s.shape[1]
    cr_pw = idx_c.shape[1]
    n_s_rows = _NW * sr_pw
    n_c_rows = _NW * cr_pw
    n_groups = cr_pw // _G
    mesh = plsc.VectorSubcoreMesh(core_axis_name="c", subcore_axis_name="s",
                                  num_cores=_NC, num_subcores=_NS)

    @functools.partial(
        pl.kernel,
        out_type=(
            jax.ShapeDtypeStruct((n_s_rows * _CHUNK, H), jnp.float32),
            jax.ShapeDtypeStruct((n_c_rows * _CHUNK, H), jnp.float32),
        ),
        mesh=mesh,
        scratch_types=[
            pltpu.VMEM((sr_pw, _CHUNK), jnp.int32),
            pltpu.VMEM((cr_pw, _CHUNK), jnp.int32),
            pltpu.VMEM((sr_pw * _CHUNK, H), jnp.float32),
            pltpu.VMEM((_G * _CHUNK, H), jnp.float32),
            pltpu.SemaphoreType.DMA,
        ],
        compiler_params=pltpu.CompilerParams(use_tc_tiling_on_sc=False),
    )
    def k(tab_s_hbm, tab_c_hbm, idx_s_hbm, idx_c_hbm, out_s, out_c,
          idx_sv, idx_cv, srow_v, crow_v, sem):
        wid = lax.axis_index("c") * _NS + lax.axis_index("s")
        pltpu.sync_copy(idx_s_hbm.at[wid], idx_sv)
        pltpu.sync_copy(idx_c_hbm.at[wid], idx_cv)

        cps = []
        for j in range(sr_pw):
            cp = pltpu.make_async_copy(
                tab_s_hbm.at[idx_sv.at[j]],
                srow_v.at[pl.ds(j * _CHUNK, _CHUNK)], sem)
            cp.start()
            cps.append(cp)
        for cp in cps:
            cp.wait()
        pltpu.sync_copy(
            srow_v, out_s.at[pl.ds(wid * sr_pw * _CHUNK, sr_pw * _CHUNK)])

        def group(g, _):
            cps = []
            for j in range(_G):
                cp = pltpu.make_async_copy(
                    tab_c_hbm.at[idx_cv.at[g * _G + j]],
                    crow_v.at[pl.ds(j * _CHUNK, _CHUNK)], sem)
                cp.start()
                cps.append(cp)
            for cp in cps:
                cp.wait()
            pltpu.sync_copy(
                crow_v,
                out_c.at[pl.ds((wid * cr_pw + g * _G) * _CHUNK, _G * _CHUNK)])
            return ()

        lax.fori_loop(0, n_groups, group, (), unroll=False)

    return k(tab_s, tab_c, idx_s, idx_c)


def _tc_project(kr, obs, cat, Wk, Wp, bk, Wo, bo):
    BT = kr.shape[0]
    R = 2048
    FK = Wk.shape[1]
    FO = Wo.shape[1]

    def body(kr_ref, obs_ref, cat_ref, wk_ref, wp_ref, bk_ref, wo_ref, bo_ref,
             outk_ref, outo_ref):
        acc = jnp.dot(kr_ref[...], wk_ref[...],
                      preferred_element_type=jnp.float32,
                      precision=lax.Precision.HIGHEST)
        acc += jnp.dot(cat_ref[...], wp_ref[...],
                       preferred_element_type=jnp.float32,
                       precision=lax.Precision.HIGHEST)
        outk_ref[...] = acc + bk_ref[...]
        outo_ref[...] = jnp.dot(obs_ref[...], wo_ref[...],
                                preferred_element_type=jnp.float32,
                                precision=lax.Precision.HIGHEST) + bo_ref[...]

    fixed = lambda i: (0, 0)
    rows = lambda i: (i, 0)
    return pl.pallas_call(
        body,
        grid=(BT // R,),
        in_specs=[
            pl.BlockSpec((R, kr.shape[1]), rows),
            pl.BlockSpec((R, obs.shape[1]), rows),
            pl.BlockSpec((R, cat.shape[1]), rows),
            pl.BlockSpec(Wk.shape, fixed),
            pl.BlockSpec(Wp.shape, fixed),
            pl.BlockSpec(bk.shape, fixed),
            pl.BlockSpec(Wo.shape, fixed),
            pl.BlockSpec(bo.shape, fixed),
        ],
        out_specs=[
            pl.BlockSpec((R, FK), rows),
            pl.BlockSpec((R, FO), rows),
        ],
        out_shape=[
            jax.ShapeDtypeStruct((BT, FK), jnp.float32),
            jax.ShapeDtypeStruct((BT, FO), jnp.float32),
        ],
    )(kr, obs, cat, Wk, Wp, bk, Wo, bo)


def kernel(static, known_real, known_categorical, observed,
           static_tables, known_cat_tables, real_W, real_b, obs_W, obs_b):
    S, V, H = static_tables.shape
    C = known_cat_tables.shape[0]
    B, T, K = known_real.shape
    O = observed.shape[2]
    F = K + C

    tab_s = static_tables.reshape(S * V, H)
    tab_c = known_cat_tables.reshape(C * V, H)
    idx_s = (static.astype(jnp.int32)
             + (jnp.arange(S, dtype=jnp.int32) * V)[None, :]
             ).reshape(_NW, -1, _CHUNK)
    idx_c = (known_categorical.astype(jnp.int32)
             + (jnp.arange(C, dtype=jnp.int32) * V)[None, None, :]
             ).reshape(_NW, -1, _CHUNK)

    srows, crows = _sc_gather(tab_s, tab_c, idx_s, idx_c)

    rw = real_W[:, 0, :]
    Wk = (rw[:, :, None]
          * jnp.eye(K, F, dtype=jnp.float32)[:, None, :]).reshape(K, H * F)
    eyeH = jnp.eye(H, dtype=jnp.float32)
    eyeCF = jnp.eye(C, F, k=K, dtype=jnp.float32)
    Wp = (eyeH[None, :, :, None]
          * eyeCF[:, None, None, :]).reshape(C * H, H * F)
    bk = jnp.concatenate(
        [real_b.T, jnp.zeros((H, C), jnp.float32)], axis=1).reshape(1, H * F)
    ow = obs_W[:, 0, :]
    Wo = (ow[:, :, None]
          * jnp.eye(O, dtype=jnp.float32)[:, None, :]).reshape(O, H * O)
    bo = obs_b.T.reshape(1, H * O)

    known_flat, obs_flat = _tc_project(
        known_real.reshape(B * T, K), observed.reshape(B * T, O),
        crows.reshape(B * T, C * H), Wk, Wp, bk, Wo, bo)

    return (srows.reshape(B, S, H),
            known_flat.reshape(B, T, H, F),
            obs_flat.reshape(B, T, H, O))


# speedup vs baseline: 2.1201x; 1.9306x over previous
import functools

import jax
import jax.numpy as jnp
from jax import lax
from jax.experimental import pallas as pl
from jax.experimental.pallas import tpu as pltpu
from jax.experimental.pallas import tpu_sc as plsc

_NC = 2
_NS = 16
_NW = _NC * _NS
_CHUNK = 128
_G = 10


def _sc_gather(tab_s, tab_c, idx_s, idx_c):
    H = tab_s.shape[-1]
    sr_pw = idx_s.shape[1]
    cr_pw = idx_c.shape[1]
    n_s_rows = _NW * sr_pw
    n_c_rows = _NW * cr_pw
    n_groups = cr_pw // _G
    mesh = plsc.VectorSubcoreMesh(core_axis_name="c", subcore_axis_name="s",
                                  num_cores=_NC, num_subcores=_NS)

    @functools.partial(
        pl.kernel,
        out_type=(
            jax.ShapeDtypeStruct((n_s_rows * _CHUNK, H), jnp.float32),
            jax.ShapeDtypeStruct((n_c_rows * _CHUNK, H), jnp.float32),
        ),
        mesh=mesh,
        scratch_types=[
            pltpu.VMEM((sr_pw, _CHUNK), jnp.int32),
            pltpu.VMEM((cr_pw, _CHUNK), jnp.int32),
            pltpu.VMEM((sr_pw * _CHUNK, H), jnp.float32),
            pltpu.VMEM((_G * _CHUNK, H), jnp.float32),
            pltpu.SemaphoreType.DMA,
        ],
        compiler_params=pltpu.CompilerParams(use_tc_tiling_on_sc=False),
    )
    def k(tab_s_hbm, tab_c_hbm, idx_s_hbm, idx_c_hbm, out_s, out_c,
          idx_sv, idx_cv, srow_v, crow_v, sem):
        wid = lax.axis_index("c") * _NS + lax.axis_index("s")
        pltpu.sync_copy(idx_s_hbm.at[wid], idx_sv)
        pltpu.sync_copy(idx_c_hbm.at[wid], idx_cv)

        cps = []
        for j in range(sr_pw):
            cp = pltpu.make_async_copy(
                tab_s_hbm.at[idx_sv.at[j]],
                srow_v.at[pl.ds(j * _CHUNK, _CHUNK)], sem)
            cp.start()
            cps.append(cp)
        for cp in cps:
            cp.wait()
        pltpu.sync_copy(
            srow_v, out_s.at[pl.ds(wid * sr_pw * _CHUNK, sr_pw * _CHUNK)])

        def group(g, _):
            cps = []
            for j in range(_G):
                cp = pltpu.make_async_copy(
                    tab_c_hbm.at[idx_cv.at[g * _G + j]],
                    crow_v.at[pl.ds(j * _CHUNK, _CHUNK)], sem)
                cp.start()
                cps.append(cp)
            for cp in cps:
                cp.wait()
            pltpu.sync_copy(
                crow_v,
                out_c.at[pl.ds((wid * cr_pw + g * _G) * _CHUNK, _G * _CHUNK)])
            return ()

        lax.fori_loop(0, n_groups, group, (), unroll=False)

    return k(tab_s, tab_c, idx_s, idx_c)


def _eye32():
    r = lax.broadcasted_iota(jnp.int32, (32, 32), 0)
    c = lax.broadcasted_iota(jnp.int32, (32, 32), 1)
    return (r == c).astype(jnp.float32)


def _rows_to_hb(x, eye, bq):
    ys = [lax.dot_general(eye, x[:, m * 32:(m + 1) * 32],
                          (((1,), (1,)), ((), ())),
                          preferred_element_type=jnp.float32,
                          precision=lax.Precision.HIGHEST)
          for m in range(4)]
    return jnp.concatenate(ys, axis=1)


def _tc_project_t(kr_t, obs_t, crows_v, rw, rb, ow_t, ob_t, C, H):
    T, K, B = kr_t.shape
    O = obs_t.shape[1]
    F = K + C
    Bq = B // 4

    def body(kr_ref, obs_ref, cat_ref, rw_ref, rb_ref, owt_ref, obt_ref,
             known_ref, obs_out_ref):
        eye = _eye32()
        kr = kr_ref[0]
        known_ref[0, 0:K] = (kr[:, None, :] * rw_ref[...][:, :, None]
                             + rb_ref[...][:, :, None])
        x = cat_ref[0]
        for c in range(C):
            xc = x[c * Bq:(c + 1) * Bq, :]
            known_ref[0, K + c] = _rows_to_hb(xc, eye, Bq)
        ob = obs_ref[0]
        obs_out_ref[0] = (ob[None, :, :] * owt_ref[...][:, :, None]
                          + obt_ref[...][:, :, None])

    rows = lambda i: (i, 0, 0)
    fixed = lambda i: (0, 0)
    return pl.pallas_call(
        body,
        grid=(T,),
        in_specs=[
            pl.BlockSpec((1, K, B), rows),
            pl.BlockSpec((1, O, B), rows),
            pl.BlockSpec((1, crows_v.shape[1], 128), rows),
            pl.BlockSpec(rw.shape, fixed),
            pl.BlockSpec(rb.shape, fixed),
            pl.BlockSpec(ow_t.shape, fixed),
            pl.BlockSpec(ob_t.shape, fixed),
        ],
        out_specs=[
            pl.BlockSpec((1, F, H, B), lambda i: (i, 0, 0, 0)),
            pl.BlockSpec((1, H, O, B), lambda i: (i, 0, 0, 0)),
        ],
        out_shape=[
            jax.ShapeDtypeStruct((T, F, H, B), jnp.float32),
            jax.ShapeDtypeStruct((T, H, O, B), jnp.float32),
        ],
    )(kr_t, obs_t, crows_v, rw, rb, ow_t, ob_t)


def _tc_static_t(srows_v, S, H, B):
    Bq = B // 4

    def body(s_ref, out_ref):
        eye = _eye32()
        out_ref[0] = _rows_to_hb(s_ref[0], eye, Bq)

    return pl.pallas_call(
        body,
        grid=(S,),
        in_specs=[pl.BlockSpec((1, srows_v.shape[1], 128),
                               lambda i: (i, 0, 0))],
        out_specs=pl.BlockSpec((1, H, B), lambda i: (i, 0, 0)),
        out_shape=jax.ShapeDtypeStruct((S, H, B), jnp.float32),
    )(srows_v)


def _permute_quads(a):
    *lead, B = a.shape
    return a.reshape(*lead, 4, B // 4).swapaxes(-1, -2).reshape(*lead, B)


def kernel(static, known_real, known_categorical, observed,
           static_tables, known_cat_tables, real_W, real_b, obs_W, obs_b):
    S, V, H = static_tables.shape
    C = known_cat_tables.shape[0]
    B, T, K = known_real.shape
    O = observed.shape[2]

    kr_t = jnp.transpose(known_real, (1, 2, 0))
    obs_t = jnp.transpose(observed, (1, 2, 0))
    kc_t = jnp.transpose(known_categorical, (1, 2, 0))
    st_t = jnp.transpose(static, (1, 0))

    tab_s = static_tables.reshape(S * V, H)
    tab_c = known_cat_tables.reshape(C * V, H)

    idx_s = (_permute_quads(st_t.astype(jnp.int32))
             + (jnp.arange(S, dtype=jnp.int32) * V)[:, None]
             ).reshape(_NW, -1, _CHUNK)
    idx_c = (_permute_quads(kc_t.astype(jnp.int32))
             + (jnp.arange(C, dtype=jnp.int32) * V)[None, :, None]
             ).reshape(_NW, -1, _CHUNK)

    srows, crows = _sc_gather(tab_s, tab_c, idx_s, idx_c)

    rw = real_W[:, 0, :]
    rb = real_b
    ow_t = obs_W[:, 0, :].T
    ob_t = obs_b.T

    crows_v = crows.reshape(T, C * B * H // 128, 128)
    srows_v = srows.reshape(S, B * H // 128, 128)

    known_t, obs_out = _tc_project_t(kr_t, obs_t, crows_v, rw, rb, ow_t, ob_t,
                                     C, H)
    static_t = _tc_static_t(srows_v, S, H, B)

    return (jnp.transpose(static_t, (2, 0, 1)),
            jnp.transpose(known_t, (3, 0, 2, 1)),
            jnp.transpose(obs_out, (3, 0, 1, 2)))
